# Initial kernel scaffold; baseline (speedup 1.0000x reference)
import jax
import jax.numpy as jnp
from jax.experimental import pallas as pl


def kernel(face_features, edge_features, edge_index, W_fp, b_fp, W_ep, b_ep, W_ni_0, W_fij_0, W_nj_0, attn_0, be_0, W_node_0, W_ni_1, W_fij_1, W_nj_1, attn_1, be_1, W_node_1, W_ef, b_ef, b_nn, W_gate, b_gate):
    raise NotImplementedError("write your pallas kernel here")


# speedup vs baseline: 4.9969x; 4.9969x over previous
import functools
import jax
import jax.numpy as jnp
from jax import lax
from jax.experimental import pallas as pl
from jax.experimental.pallas import tpu as pltpu
from jax.experimental.pallas import tpu_sc as plsc

N = 10000
NP = 10240
E = 160000
NC = 2
NS = 16
NW = NC * NS
CH = 128
N_CHUNKS = E // CH
ITERS = (N_CHUNKS + NW - 1) // NW
ROWS_PER_TILE = NP // NS
BE = 2000
GE = E // BE

_sc_mesh = plsc.VectorSubcoreMesh(core_axis_name="c", subcore_axis_name="s")


def _make_gather2(da, db):
    @functools.partial(
        pl.kernel,
        out_type=(
            jax.ShapeDtypeStruct((E, da), jnp.float32),
            jax.ShapeDtypeStruct((E, db), jnp.float32),
        ),
        mesh=_sc_mesh,
        scratch_types=[
            pltpu.VMEM((CH,), jnp.int32),
            pltpu.VMEM((CH,), jnp.int32),
            pltpu.VMEM((CH, da), jnp.float32),
            pltpu.VMEM((CH, db), jnp.float32),
            pltpu.SemaphoreType.DMA,
            pltpu.SemaphoreType.DMA,
        ],
    )
    def gather2(table_a, idx_a, table_b, idx_b, out_a, out_b,
                ia_v, ib_v, ra_v, rb_v, sem_a, sem_b):
        wid = lax.axis_index("s") * NC + lax.axis_index("c")

        def body(j):
            c = wid + j * NW

            @pl.when(c < N_CHUNKS)
            def _():
                base = c * CH
                pltpu.sync_copy(idx_a.at[pl.ds(base, CH)], ia_v)
                pltpu.sync_copy(idx_b.at[pl.ds(base, CH)], ib_v)
                cp_a = pltpu.async_copy(table_a.at[ia_v], ra_v, sem_a)
                cp_b = pltpu.async_copy(table_b.at[ib_v], rb_v, sem_b)
                cp_a.wait()
                cp_b.wait()
                pltpu.sync_copy(ra_v, out_a.at[pl.ds(base, CH)])
                pltpu.sync_copy(rb_v, out_b.at[pl.ds(base, CH)])

        pl.loop(0, ITERS)(body)

    return gather2


def _make_scatter_add(d):
    @functools.partial(
        pl.kernel,
        out_type=jax.ShapeDtypeStruct((NC, NP, d), jnp.float32),
        mesh=_sc_mesh,
        scratch_types=[
            pltpu.VMEM((1, CH), jnp.int32),
            pltpu.VMEM((CH, d), jnp.float32),
            pltpu.VMEM_SHARED((NP, d), jnp.float32),
            pltpu.SemaphoreType.DMA,
        ],
    )
    def scatter_add(vals, idx, zeros, out, idx_v, vals_v, accum, sem):
        cid = lax.axis_index("c")
        sid = lax.axis_index("s")
        wid = sid * NC + cid
        row0 = sid * ROWS_PER_TILE
        pltpu.sync_copy(zeros, accum.at[pl.ds(row0, ROWS_PER_TILE)])
        plsc.subcore_barrier()

        def body(j):
            c = wid + j * NW

            @pl.when(c < N_CHUNKS)
            def _():
                base = c * CH
                pltpu.sync_copy(idx.at[pl.ds(base, CH)], idx_v.at[0])
                pltpu.sync_copy(vals.at[pl.ds(base, CH)], vals_v)
                pltpu.sync_copy(vals_v, accum.at[idx_v.at[0]], add=True)

        pl.loop(0, ITERS)(body)
        plsc.subcore_barrier()
        pltpu.sync_copy(accum.at[pl.ds(row0, ROWS_PER_TILE)],
                        out.at[cid, pl.ds(row0, ROWS_PER_TILE)])

    return scatter_add


def _leaky_relu(x):
    return jnp.where(x >= 0, x, 0.01 * x)


def _mm(a, b):
    return jax.lax.dot_general(a, b, (((1,), (0,)), ((), ())),
                               preferred_element_type=jnp.float32)


def _tc_node0(ff_ref, wfp, bfp, wni, wnj, wnode, tsrc, tdst):
    ff = ff_ref[...]
    h0 = _mm(ff, wfp[...]) + bfp[...]
    hi = _mm(h0, wni[...])
    hm = _mm(h0, wnode[...])
    z = jnp.zeros((NP, 96), jnp.float32)
    tsrc[...] = jnp.concatenate([ff, hi, hm, z], axis=1)
    tdst[...] = jnp.concatenate([_mm(h0, wnj[...]), z[:, 0:64]], axis=1)


def _tc_edge0(g0a, g0b, ef_ref, wep, bep, wfij0, be0, attn0, wfij1, be1,
              wef2, bmat, v0, ew1):
    gff = g0a[:, 0:32]
    ghi = g0a[:, 32:96]
    ghm = g0a[:, 96:160]
    ef = ef_ref[...]
    wc0 = _mm(wep[...], wfij0[...])
    bc0 = _mm(bep[...], wfij0[...]) + be0[...]
    ew0 = _mm(ef, wc0) + bc0
    f0 = _leaky_relu(ghi + g0b[:, 0:64] + ew0)
    ex = jnp.exp(jnp.sum(f0 * attn0[...], axis=1, keepdims=True))
    ew1[...] = _mm(f0, wfij1[...]) + be1[...]
    x = jnp.concatenate([gff * ef[:, k:k + 1] for k in range(16)], axis=1)
    msg = _mm(x, wef2[...]) + _mm(gff, bmat[...])
    ci = lax.broadcasted_iota(jnp.int32, (BE, 32), 1)
    exdeg = jnp.where(ci == 0, ex, jnp.where(ci == 1, 1.0, 0.0))
    v0[...] = jnp.concatenate([ex * ghm, msg, exdeg], axis=1)


def _tc_node1(p0, bnn, wni, wnj, wnode, tsrc, tdst, ef_out):
    acc = p0[0] + p0[1]
    den = acc[:, 96:97]
    deg = acc[:, 97:98]
    h1 = acc[:, 0:64] / (den + 1e-16)
    ef_out[...] = acc[:, 64:96] / jnp.maximum(deg, 1.0) + bnn[...]
    hi = _mm(h1, wni[...])
    hm = _mm(h1, wnode[...])
    tsrc[...] = jnp.concatenate([hi, hm], axis=1)
    tdst[...] = jnp.concatenate([_mm(h1, wnj[...]),
                                 jnp.zeros((NP, 64), jnp.float32)], axis=1)


def _tc_edge1(g1a, g1b, ew1, attn1, v1):
    ghi = g1a[:, 0:64]
    ghm = g1a[:, 64:128]
    f1 = _leaky_relu(ghi + g1b[:, 0:64] + ew1[...])
    ex = jnp.exp(jnp.sum(f1 * attn1[...], axis=1, keepdims=True))
    ci = lax.broadcasted_iota(jnp.int32, (BE, 64), 1)
    exz = jnp.where(ci == 0, ex, 0.0)
    v1[...] = jnp.concatenate([ex * ghm, exz], axis=1)


def _tc_final(p1, ef_in, wgate, bgate, nf_out, gf_out):
    acc = p1[0] + p1[1]
    gf_nodes = acc[:, 0:64] / (acc[:, 64:65] + 1e-16)
    nf = jnp.concatenate([gf_nodes, ef_in[...],
                          jnp.zeros((NP, 32), jnp.float32)], axis=1)
    g = _mm(nf, wgate[...]) + bgate[...]
    m = jnp.max(g)
    valid = lax.broadcasted_iota(jnp.int32, (NP, 1), 0) < N
    p = jnp.where(valid, jnp.exp(g - m), 0.0)
    gate = p / jnp.sum(p)
    nf_out[...] = nf[0:N]
    gf_out[...] = jnp.sum(gate * nf, axis=0, keepdims=True)


def _full(shape):
    return pl.BlockSpec(shape, lambda *_: tuple(0 for _ in shape))


def kernel(face_features, edge_features, edge_index, W_fp, b_fp, W_ep, b_ep,
           W_ni_0, W_fij_0, W_nj_0, attn_0, be_0, W_node_0,
           W_ni_1, W_fij_1, W_nj_1, attn_1, be_1, W_node_1,
           W_ef, b_ef, b_nn, W_gate, b_gate):
    src = edge_index[0]
    dst = edge_index[1]
    f32 = jnp.float32
    ffp = jnp.pad(face_features, ((0, NP - N), (0, 0)))

    tsrc0, tdst0 = pl.pallas_call(
        _tc_node0,
        out_shape=(jax.ShapeDtypeStruct((NP, 256), f32),
                   jax.ShapeDtypeStruct((NP, 128), f32)),
    )(ffp, W_fp, b_fp.reshape(1, 64), W_ni_0, W_nj_0, W_node_0)

    g0a, g0b = _make_gather2(256, 128)(tsrc0, src, tdst0, dst)

    wef2 = W_ef.reshape(16, 32, 32).reshape(512, 32)
    bmat = b_ef.reshape(32, 32)
    v0, ew1 = pl.pallas_call(
        _tc_edge0,
        grid=(GE,),
        in_specs=[
            pl.BlockSpec((BE, 256), lambda i: (i, 0)),
            pl.BlockSpec((BE, 128), lambda i: (i, 0)),
            pl.BlockSpec((BE, 16), lambda i: (i, 0)),
            _full((16, 64)), _full((1, 64)), _full((64, 64)), _full((1, 64)),
            _full((1, 64)), _full((64, 64)), _full((1, 64)),
            _full((512, 32)), _full((32, 32)),
        ],
        out_specs=(pl.BlockSpec((BE, 128), lambda i: (i, 0)),
                   pl.BlockSpec((BE, 64), lambda i: (i, 0))),
        out_shape=(jax.ShapeDtypeStruct((E, 128), f32),
                   jax.ShapeDtypeStruct((E, 64), f32)),
    )(g0a, g0b, edge_features, W_ep, b_ep.reshape(1, 64), W_fij_0,
      be_0.reshape(1, 64), attn_0.reshape(1, 64), W_fij_1, be_1.reshape(1, 64),
      wef2, bmat)

    zeros = jnp.zeros((ROWS_PER_TILE, 128), f32)
    p0 = _make_scatter_add(128)(v0, dst, zeros)

    tsrc1, tdst1, ef_nodes = pl.pallas_call(
        _tc_node1,
        out_shape=(jax.ShapeDtypeStruct((NP, 128), f32),
                   jax.ShapeDtypeStruct((NP, 128), f32),
                   jax.ShapeDtypeStruct((NP, 32), f32)),
    )(p0, b_nn.reshape(1, 32), W_ni_1, W_nj_1, W_node_1)

    g1a, g1b = _make_gather2(128, 128)(tsrc1, src, tdst1, dst)

    v1 = pl.pallas_call(
        _tc_edge1,
        grid=(GE,),
        in_specs=[
            pl.BlockSpec((BE, 128), lambda i: (i, 0)),
            pl.BlockSpec((BE, 128), lambda i: (i, 0)),
            pl.BlockSpec((BE, 64), lambda i: (i, 0)),
            _full((1, 64)),
        ],
        out_specs=pl.BlockSpec((BE, 128), lambda i: (i, 0)),
        out_shape=jax.ShapeDtypeStruct((E, 128), f32),
    )(g1a, g1b, ew1, attn_1.reshape(1, 64))

    p1 = _make_scatter_add(128)(v1, dst, zeros)

    nf, gf = pl.pallas_call(
        _tc_final,
        out_shape=(jax.ShapeDtypeStruct((N, 128), f32),
                   jax.ShapeDtypeStruct((1, 128), f32)),
    )(p1, ef_nodes, W_gate, b_gate.reshape(1, 1))

    return nf, gf


# speedup vs baseline: 7.2560x; 1.4521x over previous
import functools
import jax
import jax.numpy as jnp
from jax import lax
from jax.experimental import pallas as pl
from jax.experimental.pallas import tpu as pltpu
from jax.experimental.pallas import tpu_sc as plsc

N = 10000
NP = 10240
E = 160000
NC = 2
NS = 16
NW = NC * NS
CH = 128
N_CHUNKS = E // CH
ITERS = (N_CHUNKS + NW - 1) // NW
ROWS_PER_TILE = NP // NS
BE = 2000
GE = E // BE

_sc_mesh = plsc.VectorSubcoreMesh(core_axis_name="c", subcore_axis_name="s")


def _make_gather2(da, db):
    @functools.partial(
        pl.kernel,
        out_type=(
            jax.ShapeDtypeStruct((E, da), jnp.float32),
            jax.ShapeDtypeStruct((E, db), jnp.float32),
        ),
        mesh=_sc_mesh,
        scratch_types=[
            pltpu.VMEM((CH,), jnp.int32),
            pltpu.VMEM((CH,), jnp.int32),
            pltpu.VMEM((CH, da), jnp.float32),
            pltpu.VMEM((CH, db), jnp.float32),
            pltpu.SemaphoreType.DMA,
            pltpu.SemaphoreType.DMA,
        ],
    )
    def gather2(table_a, idx_a, table_b, idx_b, out_a, out_b,
                ia_v, ib_v, ra_v, rb_v, sem_a, sem_b):
        wid = lax.axis_index("s") * NC + lax.axis_index("c")

        def body(j):
            c = wid + j * NW

            @pl.when(c < N_CHUNKS)
            def _():
                base = c * CH
                pltpu.sync_copy(idx_a.at[pl.ds(base, CH)], ia_v)
                pltpu.sync_copy(idx_b.at[pl.ds(base, CH)], ib_v)
                cp_a = pltpu.async_copy(table_a.at[ia_v], ra_v, sem_a)
                cp_b = pltpu.async_copy(table_b.at[ib_v], rb_v, sem_b)
                cp_a.wait()
                cp_b.wait()
                pltpu.sync_copy(ra_v, out_a.at[pl.ds(base, CH)])
                pltpu.sync_copy(rb_v, out_b.at[pl.ds(base, CH)])

        pl.loop(0, ITERS)(body)

    return gather2


def _make_scatter_add(d):
    @functools.partial(
        pl.kernel,
        out_type=jax.ShapeDtypeStruct((NC, NP, d), jnp.float32),
        mesh=_sc_mesh,
        scratch_types=[
            pltpu.VMEM((1, CH), jnp.int32),
            pltpu.VMEM((CH, d), jnp.float32),
            pltpu.VMEM_SHARED((NP, d), jnp.float32),
            pltpu.SemaphoreType.DMA,
        ],
    )
    def scatter_add(vals, idx, zeros, out, idx_v, vals_v, accum, sem):
        cid = lax.axis_index("c")
        sid = lax.axis_index("s")
        wid = sid * NC + cid
        row0 = sid * ROWS_PER_TILE
        pltpu.sync_copy(zeros, accum.at[pl.ds(row0, ROWS_PER_TILE)])
        plsc.subcore_barrier()

        def body(j):
            c = wid + j * NW

            @pl.when(c < N_CHUNKS)
            def _():
                base = c * CH
                pltpu.sync_copy(idx.at[pl.ds(base, CH)], idx_v.at[0])
                pltpu.sync_copy(vals.at[pl.ds(base, CH)], vals_v)
                pltpu.sync_copy(vals_v, accum.at[idx_v.at[0]], add=True)

        pl.loop(0, ITERS)(body)
        plsc.subcore_barrier()
        pltpu.sync_copy(accum.at[pl.ds(row0, ROWS_PER_TILE)],
                        out.at[cid, pl.ds(row0, ROWS_PER_TILE)])

    return scatter_add


def _leaky_relu(x):
    return jnp.where(x >= 0, x, 0.01 * x)


def _mm(a, b):
    return jax.lax.dot_general(a, b, (((1,), (0,)), ((), ())),
                               preferred_element_type=jnp.float32)


def _tc_node0(ff_ref, wfp, bfp, wni, wnj, wnode, tsrc, tdst):
    ff = ff_ref[...]
    h0 = _mm(ff, wfp[...]) + bfp[...]
    hi = _mm(h0, wni[...])
    hm = _mm(h0, wnode[...])
    z = jnp.zeros((NP, 96), jnp.float32)
    tsrc[...] = jnp.concatenate([ff, hi, hm, z], axis=1)
    tdst[...] = jnp.concatenate([_mm(h0, wnj[...]), z[:, 0:64]], axis=1)


def _tc_edge0(g0a, g0b, ef_ref, wep, bep, wfij0, be0, attn0, wfij1, be1,
              wef2, bmat, rmask, tmask, v0, ew1):
    gff = g0a[:, 0:32]
    ghi = g0a[:, 32:96]
    ghm = g0a[:, 96:160]
    ef = ef_ref[...]
    wc0 = _mm(wep[...], wfij0[...])
    bc0 = _mm(bep[...], wfij0[...]) + be0[...]
    ew0 = _mm(ef, wc0) + bc0
    f0 = _leaky_relu(ghi + g0b[:, 0:64] + ew0)
    ex = jnp.exp(jnp.sum(f0 * attn0[...], axis=1, keepdims=True))
    ew1[...] = _mm(f0, wfij1[...]) + be1[...]
    x = _mm(ef, rmask[...]) * _mm(gff, tmask[...])
    msg = _mm(x, wef2[...]) + _mm(gff, bmat[...])
    ci = lax.broadcasted_iota(jnp.int32, (BE, 32), 1)
    exdeg = jnp.where(ci == 0, ex, jnp.where(ci == 1, 1.0, 0.0))
    v0[...] = jnp.concatenate([ex * ghm, msg, exdeg], axis=1)


def _tc_node1(p0, bnn, wni, wnj, wnode, tsrc, tdst, ef_out):
    acc = p0[0] + p0[1]
    den = acc[:, 96:97]
    deg = acc[:, 97:98]
    h1 = acc[:, 0:64] / (den + 1e-16)
    ef_out[...] = acc[:, 64:96] / jnp.maximum(deg, 1.0) + bnn[...]
    hi = _mm(h1, wni[...])
    hm = _mm(h1, wnode[...])
    tsrc[...] = jnp.concatenate([hi, hm], axis=1)
    tdst[...] = jnp.concatenate([_mm(h1, wnj[...]),
                                 jnp.zeros((NP, 64), jnp.float32)], axis=1)


def _tc_edge1(g1a, g1b, ew1, attn1, v1):
    ghi = g1a[:, 0:64]
    ghm = g1a[:, 64:128]
    f1 = _leaky_relu(ghi + g1b[:, 0:64] + ew1[...])
    ex = jnp.exp(jnp.sum(f1 * attn1[...], axis=1, keepdims=True))
    ci = lax.broadcasted_iota(jnp.int32, (BE, 64), 1)
    exz = jnp.where(ci == 0, ex, 0.0)
    v1[...] = jnp.concatenate([ex * ghm, exz], axis=1)


def _tc_final(p1, ef_in, wgate, bgate, nf_out, gf_out):
    acc = p1[0] + p1[1]
    gf_nodes = acc[:, 0:64] / (acc[:, 64:65] + 1e-16)
    nf = jnp.concatenate([gf_nodes, ef_in[...],
                          jnp.zeros((NP, 32), jnp.float32)], axis=1)
    g = _mm(nf, wgate[...]) + bgate[...]
    m = jnp.max(g)
    valid = lax.broadcasted_iota(jnp.int32, (NP, 1), 0) < N
    p = jnp.where(valid, jnp.exp(g - m), 0.0)
    gate = p / jnp.sum(p)
    nf_out[...] = nf[0:N]
    gf_out[...] = jnp.sum(gate * nf, axis=0, keepdims=True)


def _full(shape):
    return pl.BlockSpec(shape, lambda *_: tuple(0 for _ in shape))


def kernel(face_features, edge_features, edge_index, W_fp, b_fp, W_ep, b_ep,
           W_ni_0, W_fij_0, W_nj_0, attn_0, be_0, W_node_0,
           W_ni_1, W_fij_1, W_nj_1, attn_1, be_1, W_node_1,
           W_ef, b_ef, b_nn, W_gate, b_gate):
    src = edge_index[0]
    dst = edge_index[1]
    f32 = jnp.float32
    ffp = jnp.pad(face_features, ((0, NP - N), (0, 0)))

    tsrc0, tdst0 = pl.pallas_call(
        _tc_node0,
        out_shape=(jax.ShapeDtypeStruct((NP, 256), f32),
                   jax.ShapeDtypeStruct((NP, 128), f32)),
    )(ffp, W_fp, b_fp.reshape(1, 64), W_ni_0, W_nj_0, W_node_0)

    g0a, g0b = _make_gather2(256, 128)(tsrc0, src, tdst0, dst)

    wef2 = W_ef.reshape(16, 32, 32).reshape(512, 32)
    bmat = b_ef.reshape(32, 32)
    v0_call = pl.pallas_call(
        _tc_edge0,
        grid=(GE,),
        in_specs=[
            pl.BlockSpec((BE, 256), lambda i: (i, 0)),
            pl.BlockSpec((BE, 128), lambda i: (i, 0)),
            pl.BlockSpec((BE, 16), lambda i: (i, 0)),
            _full((16, 64)), _full((1, 64)), _full((64, 64)), _full((1, 64)),
            _full((1, 64)), _full((64, 64)), _full((1, 64)),
            _full((512, 32)), _full((32, 32)),
            _full((16, 512)), _full((32, 512)),
        ],
        out_specs=(pl.BlockSpec((BE, 128), lambda i: (i, 0)),
                   pl.BlockSpec((BE, 64), lambda i: (i, 0))),
        out_shape=(jax.ShapeDtypeStruct((E, 128), f32),
                   jax.ShapeDtypeStruct((E, 64), f32)),
    )
    rmask = jnp.kron(jnp.eye(16, dtype=f32), jnp.ones((1, 32), f32))
    tmask = jnp.tile(jnp.eye(32, dtype=f32), (1, 16))
    v0, ew1 = v0_call(g0a, g0b, edge_features, W_ep, b_ep.reshape(1, 64),
                      W_fij_0, be_0.reshape(1, 64), attn_0.reshape(1, 64),
                      W_fij_1, be_1.reshape(1, 64), wef2, bmat, rmask, tmask)

    zeros = jnp.zeros((ROWS_PER_TILE, 128), f32)
    p0 = _make_scatter_add(128)(v0, dst, zeros)

    tsrc1, tdst1, ef_nodes = pl.pallas_call(
        _tc_node1,
        out_shape=(jax.ShapeDtypeStruct((NP, 128), f32),
                   jax.ShapeDtypeStruct((NP, 128), f32),
                   jax.ShapeDtypeStruct((NP, 32), f32)),
    )(p0, b_nn.reshape(1, 32), W_ni_1, W_nj_1, W_node_1)

    g1a, g1b = _make_gather2(128, 128)(tsrc1, src, tdst1, dst)

    v1 = pl.pallas_call(
        _tc_edge1,
        grid=(GE,),
        in_specs=[
            pl.BlockSpec((BE, 128), lambda i: (i, 0)),
            pl.BlockSpec((BE, 128), lambda i: (i, 0)),
            pl.BlockSpec((BE, 64), lambda i: (i, 0)),
            _full((1, 64)),
        ],
        out_specs=pl.BlockSpec((BE, 128), lambda i: (i, 0)),
        out_shape=jax.ShapeDtypeStruct((E, 128), f32),
    )(g1a, g1b, ew1, attn_1.reshape(1, 64))

    p1 = _make_scatter_add(128)(v1, dst, zeros)

    nf, gf = pl.pallas_call(
        _tc_final,
        out_shape=(jax.ShapeDtypeStruct((N, 128), f32),
                   jax.ShapeDtypeStruct((1, 128), f32)),
    )(p1, ef_nodes, W_gate, b_gate.reshape(1, 1))

    return nf, gf


# speedup vs baseline: 8.3359x; 1.1488x over previous
import functools
import jax
import jax.numpy as jnp
from jax import lax
from jax.experimental import pallas as pl
from jax.experimental.pallas import tpu as pltpu
from jax.experimental.pallas import tpu_sc as plsc

N = 10000
NP = 10240
E = 160000
NC = 2
NS = 16
NW = NC * NS
CH = 128
N_CHUNKS = E // CH
ITERS = (N_CHUNKS + NW - 1) // NW
ROWS_PER_TILE = NP // NS
BE = 2000
GE = E // BE

_sc_mesh = plsc.VectorSubcoreMesh(core_axis_name="c", subcore_axis_name="s")


def _make_gather2(da, db):
    @functools.partial(
        pl.kernel,
        out_type=(
            jax.ShapeDtypeStruct((E, da), jnp.float32),
            jax.ShapeDtypeStruct((E, db), jnp.float32),
        ),
        mesh=_sc_mesh,
        scratch_types=[
            pltpu.VMEM((2, CH), jnp.int32),
            pltpu.VMEM((2, CH), jnp.int32),
            pltpu.VMEM((2, CH, da), jnp.float32),
            pltpu.VMEM((2, CH, db), jnp.float32),
            pltpu.SemaphoreType.DMA,
            pltpu.SemaphoreType.DMA,
            pltpu.SemaphoreType.DMA,
            pltpu.SemaphoreType.DMA,
        ],
    )
    def gather2(table_a, idx_a, table_b, idx_b, out_a, out_b,
                ia_v, ib_v, ra_v, rb_v, sa0, sb0, sa1, sb1):
        wid = lax.axis_index("s") * NC + lax.axis_index("c")
        sems = ((sa0, sb0), (sa1, sb1))

        def body(jo):
            for t in range(2):
                c = wid + (2 * jo + t) * NW

                @pl.when(c < N_CHUNKS)
                def _(c=c, t=t):
                    base = c * CH
                    pltpu.sync_copy(idx_a.at[pl.ds(base, CH)], ia_v.at[t])
                    pltpu.sync_copy(idx_b.at[pl.ds(base, CH)], ib_v.at[t])
                    pltpu.async_copy(table_a.at[ia_v.at[t]], ra_v.at[t],
                                     sems[t][0])
                    pltpu.async_copy(table_b.at[ib_v.at[t]], rb_v.at[t],
                                     sems[t][1])

            for t in range(2):
                c = wid + (2 * jo + t) * NW

                @pl.when(c < N_CHUNKS)
                def _(c=c, t=t):
                    base = c * CH
                    pltpu.make_async_copy(table_a.at[ia_v.at[t]], ra_v.at[t],
                                          sems[t][0]).wait()
                    pltpu.make_async_copy(table_b.at[ib_v.at[t]], rb_v.at[t],
                                          sems[t][1]).wait()
                    pltpu.sync_copy(ra_v.at[t], out_a.at[pl.ds(base, CH)])
                    pltpu.sync_copy(rb_v.at[t], out_b.at[pl.ds(base, CH)])

        pl.loop(0, (ITERS + 1) // 2)(body)

    return gather2


def _make_scatter_add(d):
    @functools.partial(
        pl.kernel,
        out_type=jax.ShapeDtypeStruct((NC, NP, d), jnp.float32),
        mesh=_sc_mesh,
        scratch_types=[
            pltpu.VMEM((2, CH), jnp.int32),
            pltpu.VMEM((2, CH, d), jnp.float32),
            pltpu.VMEM_SHARED((NP, d), jnp.float32),
            pltpu.SemaphoreType.DMA,
            pltpu.SemaphoreType.DMA,
        ],
    )
    def scatter_add(vals, idx, zeros, out, idx_v, vals_v, accum, sv0, sv1):
        cid = lax.axis_index("c")
        sid = lax.axis_index("s")
        wid = sid * NC + cid
        row0 = sid * ROWS_PER_TILE
        sems = (sv0, sv1)
        pltpu.sync_copy(zeros, accum.at[pl.ds(row0, ROWS_PER_TILE)])
        plsc.subcore_barrier()

        def body(jo):
            for t in range(2):
                c = wid + (2 * jo + t) * NW

                @pl.when(c < N_CHUNKS)
                def _(c=c, t=t):
                    base = c * CH
                    pltpu.sync_copy(idx.at[pl.ds(base, CH)], idx_v.at[t])
                    pltpu.async_copy(vals.at[pl.ds(base, CH)], vals_v.at[t],
                                     sems[t])

            for t in range(2):
                c = wid + (2 * jo + t) * NW

                @pl.when(c < N_CHUNKS)
                def _(c=c, t=t):
                    pltpu.make_async_copy(vals.at[pl.ds(c * CH, CH)],
                                          vals_v.at[t], sems[t]).wait()
                    pltpu.sync_copy(vals_v.at[t], accum.at[idx_v.at[t]],
                                    add=True)

        pl.loop(0, (ITERS + 1) // 2)(body)
        plsc.subcore_barrier()
        pltpu.sync_copy(accum.at[pl.ds(row0, ROWS_PER_TILE)],
                        out.at[cid, pl.ds(row0, ROWS_PER_TILE)])

    return scatter_add


def _leaky_relu(x):
    return jnp.where(x >= 0, x, 0.01 * x)


def _mm(a, b):
    return jax.lax.dot_general(a, b, (((1,), (0,)), ((), ())),
                               preferred_element_type=jnp.float32)


def _tc_node0(ff_ref, wfp, bfp, wni, wnj, wnode, tsrc, tdst):
    ff = ff_ref[...]
    h0 = _mm(ff, wfp[...]) + bfp[...]
    hi = _mm(h0, wni[...])
    hm = _mm(h0, wnode[...])
    z = jnp.zeros((NP, 96), jnp.float32)
    tsrc[...] = jnp.concatenate([ff, hi, hm, z], axis=1)
    tdst[...] = jnp.concatenate([_mm(h0, wnj[...]), z[:, 0:64]], axis=1)


def _tc_edge0(g0a, g0b, ef_ref, wep, bep, wfij0, be0, attn0, wfij1, be1,
              wef2, bmat, rmask, tmask, v0, ew1):
    gff = g0a[:, 0:32]
    ghi = g0a[:, 32:96]
    ghm = g0a[:, 96:160]
    ef = ef_ref[...]
    wc0 = _mm(wep[...], wfij0[...])
    bc0 = _mm(bep[...], wfij0[...]) + be0[...]
    ew0 = _mm(ef, wc0) + bc0
    f0 = _leaky_relu(ghi + g0b[:, 0:64] + ew0)
    ex = jnp.exp(jnp.sum(f0 * attn0[...], axis=1, keepdims=True))
    ew1[...] = _mm(f0, wfij1[...]) + be1[...]
    x = _mm(ef, rmask[...]) * _mm(gff, tmask[...])
    msg = _mm(x, wef2[...]) + _mm(gff, bmat[...])
    ci = lax.broadcasted_iota(jnp.int32, (BE, 32), 1)
    exdeg = jnp.where(ci == 0, ex, jnp.where(ci == 1, 1.0, 0.0))
    v0[...] = jnp.concatenate([ex * ghm, msg, exdeg], axis=1)


def _tc_node1(p0, bnn, wni, wnj, wnode, tsrc, tdst, ef_out):
    acc = p0[0] + p0[1]
    den = acc[:, 96:97]
    deg = acc[:, 97:98]
    h1 = acc[:, 0:64] / (den + 1e-16)
    ef_out[...] = acc[:, 64:96] / jnp.maximum(deg, 1.0) + bnn[...]
    hi = _mm(h1, wni[...])
    hm = _mm(h1, wnode[...])
    tsrc[...] = jnp.concatenate([hi, hm], axis=1)
    tdst[...] = jnp.concatenate([_mm(h1, wnj[...]),
                                 jnp.zeros((NP, 64), jnp.float32)], axis=1)


def _tc_edge1(g1a, g1b, ew1, attn1, v1):
    ghi = g1a[:, 0:64]
    ghm = g1a[:, 64:128]
    f1 = _leaky_relu(ghi + g1b[:, 0:64] + ew1[...])
    ex = jnp.exp(jnp.sum(f1 * attn1[...], axis=1, keepdims=True))
    ci = lax.broadcasted_iota(jnp.int32, (BE, 64), 1)
    exz = jnp.where(ci == 0, ex, 0.0)
    v1[...] = jnp.concatenate([ex * ghm, exz], axis=1)


def _tc_final(p1, ef_in, wgate, bgate, nf_out, gf_out):
    acc = p1[0] + p1[1]
    gf_nodes = acc[:, 0:64] / (acc[:, 64:65] + 1e-16)
    nf = jnp.concatenate([gf_nodes, ef_in[...],
                          jnp.zeros((NP, 32), jnp.float32)], axis=1)
    g = _mm(nf, wgate[...]) + bgate[...]
    m = jnp.max(g)
    valid = lax.broadcasted_iota(jnp.int32, (NP, 1), 0) < N
    p = jnp.where(valid, jnp.exp(g - m), 0.0)
    gate = p / jnp.sum(p)
    nf_out[...] = nf[0:N]
    gf_out[...] = jnp.sum(gate * nf, axis=0, keepdims=True)


def _full(shape):
    return pl.BlockSpec(shape, lambda *_: tuple(0 for _ in shape))


def kernel(face_features, edge_features, edge_index, W_fp, b_fp, W_ep, b_ep,
           W_ni_0, W_fij_0, W_nj_0, attn_0, be_0, W_node_0,
           W_ni_1, W_fij_1, W_nj_1, attn_1, be_1, W_node_1,
           W_ef, b_ef, b_nn, W_gate, b_gate):
    src = edge_index[0]
    dst = edge_index[1]
    f32 = jnp.float32
    ffp = jnp.pad(face_features, ((0, NP - N), (0, 0)))

    tsrc0, tdst0 = pl.pallas_call(
        _tc_node0,
        out_shape=(jax.ShapeDtypeStruct((NP, 256), f32),
                   jax.ShapeDtypeStruct((NP, 128), f32)),
    )(ffp, W_fp, b_fp.reshape(1, 64), W_ni_0, W_nj_0, W_node_0)

    g0a, g0b = _make_gather2(256, 128)(tsrc0, src, tdst0, dst)

    wef2 = W_ef.reshape(16, 32, 32).reshape(512, 32)
    bmat = b_ef.reshape(32, 32)
    v0_call = pl.pallas_call(
        _tc_edge0,
        grid=(GE,),
        in_specs=[
            pl.BlockSpec((BE, 256), lambda i: (i, 0)),
            pl.BlockSpec((BE, 128), lambda i: (i, 0)),
            pl.BlockSpec((BE, 16), lambda i: (i, 0)),
            _full((16, 64)), _full((1, 64)), _full((64, 64)), _full((1, 64)),
            _full((1, 64)), _full((64, 64)), _full((1, 64)),
            _full((512, 32)), _full((32, 32)),
            _full((16, 512)), _full((32, 512)),
        ],
        out_specs=(pl.BlockSpec((BE, 128), lambda i: (i, 0)),
                   pl.BlockSpec((BE, 64), lambda i: (i, 0))),
        out_shape=(jax.ShapeDtypeStruct((E, 128), f32),
                   jax.ShapeDtypeStruct((E, 64), f32)),
    )
    rmask = jnp.kron(jnp.eye(16, dtype=f32), jnp.ones((1, 32), f32))
    tmask = jnp.tile(jnp.eye(32, dtype=f32), (1, 16))
    v0, ew1 = v0_call(g0a, g0b, edge_features, W_ep, b_ep.reshape(1, 64),
                      W_fij_0, be_0.reshape(1, 64), attn_0.reshape(1, 64),
                      W_fij_1, be_1.reshape(1, 64), wef2, bmat, rmask, tmask)

    zeros = jnp.zeros((ROWS_PER_TILE, 128), f32)
    p0 = _make_scatter_add(128)(v0, dst, zeros)

    tsrc1, tdst1, ef_nodes = pl.pallas_call(
        _tc_node1,
        out_shape=(jax.ShapeDtypeStruct((NP, 128), f32),
                   jax.ShapeDtypeStruct((NP, 128), f32),
                   jax.ShapeDtypeStruct((NP, 32), f32)),
    )(p0, b_nn.reshape(1, 32), W_ni_1, W_nj_1, W_node_1)

    g1a, g1b = _make_gather2(128, 128)(tsrc1, src, tdst1, dst)

    v1 = pl.pallas_call(
        _tc_edge1,
        grid=(GE,),
        in_specs=[
            pl.BlockSpec((BE, 128), lambda i: (i, 0)),
            pl.BlockSpec((BE, 128), lambda i: (i, 0)),
            pl.BlockSpec((BE, 64), lambda i: (i, 0)),
            _full((1, 64)),
        ],
        out_specs=pl.BlockSpec((BE, 128), lambda i: (i, 0)),
        out_shape=jax.ShapeDtypeStruct((E, 128), f32),
    )(g1a, g1b, ew1, attn_1.reshape(1, 64))

    p1 = _make_scatter_add(128)(v1, dst, zeros)

    nf, gf = pl.pallas_call(
        _tc_final,
        out_shape=(jax.ShapeDtypeStruct((N, 128), f32),
                   jax.ShapeDtypeStruct((1, 128), f32)),
    )(p1, ef_nodes, W_gate, b_gate.reshape(1, 1))

    return nf, gf


# speedup vs baseline: 8.7738x; 1.0525x over previous
import functools
import jax
import jax.numpy as jnp
from jax import lax
from jax.experimental import pallas as pl
from jax.experimental.pallas import tpu as pltpu
from jax.experimental.pallas import tpu_sc as plsc

N = 10000
NP = 10240
E = 160000
NC = 2
NS = 16
NW = NC * NS
CH = 128
N_CHUNKS = E // CH
ITERS = (N_CHUNKS + NW - 1) // NW
ROWS_PER_TILE = NP // NS
BE = 2000
GE = E // BE

_sc_mesh = plsc.VectorSubcoreMesh(core_axis_name="c", subcore_axis_name="s")


def _make_gather2(da, db):
    @functools.partial(
        pl.kernel,
        out_type=(
            jax.ShapeDtypeStruct((E, da), jnp.float32),
            jax.ShapeDtypeStruct((E, db), jnp.float32),
        ),
        mesh=_sc_mesh,
        scratch_types=[
            pltpu.VMEM((2, CH), jnp.int32),
            pltpu.VMEM((2, CH), jnp.int32),
            pltpu.VMEM((2, CH, da), jnp.float32),
            pltpu.VMEM((2, CH, db), jnp.float32),
            pltpu.SemaphoreType.DMA,
            pltpu.SemaphoreType.DMA,
            pltpu.SemaphoreType.DMA,
            pltpu.SemaphoreType.DMA,
        ],
    )
    def gather2(table_a, idx_a, table_b, idx_b, out_a, out_b,
                ia_v, ib_v, ra_v, rb_v, sa0, sb0, sa1, sb1):
        wid = lax.axis_index("s") * NC + lax.axis_index("c")
        sems = ((sa0, sb0), (sa1, sb1))

        def body(jo):
            for t in range(2):
                c = wid + (2 * jo + t) * NW

                @pl.when(c < N_CHUNKS)
                def _(c=c, t=t):
                    base = c * CH
                    pltpu.sync_copy(idx_a.at[pl.ds(base, CH)], ia_v.at[t])
                    pltpu.sync_copy(idx_b.at[pl.ds(base, CH)], ib_v.at[t])
                    pltpu.async_copy(table_a.at[ia_v.at[t]], ra_v.at[t],
                                     sems[t][0])
                    pltpu.async_copy(table_b.at[ib_v.at[t]], rb_v.at[t],
                                     sems[t][1])

            for t in range(2):
                c = wid + (2 * jo + t) * NW

                @pl.when(c < N_CHUNKS)
                def _(c=c, t=t):
                    base = c * CH
                    pltpu.make_async_copy(table_a.at[ia_v.at[t]], ra_v.at[t],
                                          sems[t][0]).wait()
                    pltpu.make_async_copy(table_b.at[ib_v.at[t]], rb_v.at[t],
                                          sems[t][1]).wait()
                    pltpu.sync_copy(ra_v.at[t], out_a.at[pl.ds(base, CH)])
                    pltpu.sync_copy(rb_v.at[t], out_b.at[pl.ds(base, CH)])

        pl.loop(0, (ITERS + 1) // 2)(body)

    return gather2


def _make_scatter_add(d):
    @functools.partial(
        pl.kernel,
        out_type=jax.ShapeDtypeStruct((NC, NP, d), jnp.float32),
        mesh=_sc_mesh,
        scratch_types=[
            pltpu.VMEM((2, CH), jnp.int32),
            pltpu.VMEM((2, CH, d), jnp.float32),
            pltpu.VMEM_SHARED((NP, d), jnp.float32),
            pltpu.SemaphoreType.DMA,
            pltpu.SemaphoreType.DMA,
        ],
    )
    def scatter_add(vals, idx, zeros, out, idx_v, vals_v, accum, sv0, sv1):
        cid = lax.axis_index("c")
        sid = lax.axis_index("s")
        wid = sid * NC + cid
        row0 = sid * ROWS_PER_TILE
        sems = (sv0, sv1)
        pltpu.sync_copy(zeros, accum.at[pl.ds(row0, ROWS_PER_TILE)])
        plsc.subcore_barrier()

        def body(jo):
            for t in range(2):
                c = wid + (2 * jo + t) * NW

                @pl.when(c < N_CHUNKS)
                def _(c=c, t=t):
                    base = c * CH
                    pltpu.sync_copy(idx.at[pl.ds(base, CH)], idx_v.at[t])
                    pltpu.async_copy(vals.at[pl.ds(base, CH)], vals_v.at[t],
                                     sems[t])

            for t in range(2):
                c = wid + (2 * jo + t) * NW

                @pl.when(c < N_CHUNKS)
                def _(c=c, t=t):
                    pltpu.make_async_copy(vals.at[pl.ds(c * CH, CH)],
                                          vals_v.at[t], sems[t]).wait()
                    pltpu.sync_copy(vals_v.at[t], accum.at[idx_v.at[t]],
                                    add=True)

        pl.loop(0, (ITERS + 1) // 2)(body)
        plsc.subcore_barrier()
        pltpu.sync_copy(accum.at[pl.ds(row0, ROWS_PER_TILE)],
                        out.at[cid, pl.ds(row0, ROWS_PER_TILE)])

    return scatter_add


def _pack_bf16_words(lo, hi):
    bl = jax.lax.bitcast_convert_type(lo, jnp.uint32)
    bh = jax.lax.bitcast_convert_type(hi, jnp.uint32)
    bl = (bl + jnp.uint32(0x8000)) >> 16
    bh = (bh + jnp.uint32(0x8000)) & jnp.uint32(0xFFFF0000)
    return jax.lax.bitcast_convert_type(bh | bl, jnp.float32)


def _unpack_bf16_words(w):
    b = jax.lax.bitcast_convert_type(w, jnp.uint32)
    lo = jax.lax.bitcast_convert_type(b << 16, jnp.float32)
    hi = jax.lax.bitcast_convert_type(b & jnp.uint32(0xFFFF0000), jnp.float32)
    return lo, hi


def _leaky_relu(x):
    return jnp.where(x >= 0, x, 0.01 * x)


def _mm(a, b):
    return jax.lax.dot_general(a, b, (((1,), (0,)), ((), ())),
                               preferred_element_type=jnp.float32)


def _tc_node0(ff_ref, wfp, bfp, wni, wnj, wnode, tsrc, tdst):
    ff = ff_ref[...]
    h0 = _mm(ff, wfp[...]) + bfp[...]
    hi = _mm(h0, wni[...])
    hm = _mm(h0, wnode[...])
    z = jnp.zeros((NP, 32), jnp.float32)
    lo = jnp.concatenate([ff, hi, z], axis=1)
    hi2 = jnp.concatenate([hm, z, z], axis=1)
    tsrc[...] = _pack_bf16_words(lo, hi2)
    tdst[...] = jnp.concatenate([_mm(h0, wnj[...]), z, z], axis=1)


def _tc_edge0(g0a, g0b, ef_ref, wep, bep, wfij0, be0, attn0, wfij1, be1,
              wef2, bmat, rmask, tmask, v0, ew1):
    lo, hi2 = _unpack_bf16_words(g0a[...])
    gff = lo[:, 0:32]
    ghi = lo[:, 32:96]
    ghm = hi2[:, 0:64]
    ef = ef_ref[...]
    wc0 = _mm(wep[...], wfij0[...])
    bc0 = _mm(bep[...], wfij0[...]) + be0[...]
    ew0 = _mm(ef, wc0) + bc0
    f0 = _leaky_relu(ghi + g0b[:, 0:64] + ew0)
    ex = jnp.exp(jnp.sum(f0 * attn0[...], axis=1, keepdims=True))
    ew1[...] = _mm(f0, wfij1[...]) + be1[...]
    x = _mm(ef, rmask[...]) * _mm(gff, tmask[...])
    msg = _mm(x, wef2[...]) + _mm(gff, bmat[...])
    ci = lax.broadcasted_iota(jnp.int32, (BE, 32), 1)
    exdeg = jnp.where(ci == 0, ex, jnp.where(ci == 1, 1.0, 0.0))
    v0[...] = jnp.concatenate([ex * ghm, msg, exdeg], axis=1)


def _tc_node1(p0, bnn, wni, wnj, wnode, tsrc, tdst, ef_out):
    acc = p0[0] + p0[1]
    den = acc[:, 96:97]
    deg = acc[:, 97:98]
    h1 = acc[:, 0:64] / (den + 1e-16)
    ef_out[...] = acc[:, 64:96] / jnp.maximum(deg, 1.0) + bnn[...]
    hi = _mm(h1, wni[...])
    hm = _mm(h1, wnode[...])
    tsrc[...] = jnp.concatenate([hi, hm], axis=1)
    tdst[...] = jnp.concatenate([_mm(h1, wnj[...]),
                                 jnp.zeros((NP, 64), jnp.float32)], axis=1)


def _tc_edge1(g1a, g1b, ew1, attn1, v1):
    ghi = g1a[:, 0:64]
    ghm = g1a[:, 64:128]
    f1 = _leaky_relu(ghi + g1b[:, 0:64] + ew1[...])
    ex = jnp.exp(jnp.sum(f1 * attn1[...], axis=1, keepdims=True))
    ci = lax.broadcasted_iota(jnp.int32, (BE, 64), 1)
    exz = jnp.where(ci == 0, ex, 0.0)
    v1[...] = jnp.concatenate([ex * ghm, exz], axis=1)


def _tc_final(p1, ef_in, wgate, bgate, nf_out, gf_out):
    acc = p1[0] + p1[1]
    gf_nodes = acc[:, 0:64] / (acc[:, 64:65] + 1e-16)
    nf = jnp.concatenate([gf_nodes, ef_in[...],
                          jnp.zeros((NP, 32), jnp.float32)], axis=1)
    g = _mm(nf, wgate[...]) + bgate[...]
    m = jnp.max(g)
    valid = lax.broadcasted_iota(jnp.int32, (NP, 1), 0) < N
    p = jnp.where(valid, jnp.exp(g - m), 0.0)
    gate = p / jnp.sum(p)
    nf_out[...] = nf[0:N]
    gf_out[...] = jnp.sum(gate * nf, axis=0, keepdims=True)


def _full(shape):
    return pl.BlockSpec(shape, lambda *_: tuple(0 for _ in shape))


def kernel(face_features, edge_features, edge_index, W_fp, b_fp, W_ep, b_ep,
           W_ni_0, W_fij_0, W_nj_0, attn_0, be_0, W_node_0,
           W_ni_1, W_fij_1, W_nj_1, attn_1, be_1, W_node_1,
           W_ef, b_ef, b_nn, W_gate, b_gate):
    src = edge_index[0]
    dst = edge_index[1]
    f32 = jnp.float32
    ffp = jnp.pad(face_features, ((0, NP - N), (0, 0)))

    tsrc0, tdst0 = pl.pallas_call(
        _tc_node0,
        out_shape=(jax.ShapeDtypeStruct((NP, 128), f32),
                   jax.ShapeDtypeStruct((NP, 128), f32)),
    )(ffp, W_fp, b_fp.reshape(1, 64), W_ni_0, W_nj_0, W_node_0)

    g0a, g0b = _make_gather2(128, 128)(tsrc0, src, tdst0, dst)

    wef2 = W_ef.reshape(16, 32, 32).reshape(512, 32)
    bmat = b_ef.reshape(32, 32)
    v0_call = pl.pallas_call(
        _tc_edge0,
        grid=(GE,),
        in_specs=[
            pl.BlockSpec((BE, 128), lambda i: (i, 0)),
            pl.BlockSpec((BE, 128), lambda i: (i, 0)),
            pl.BlockSpec((BE, 16), lambda i: (i, 0)),
            _full((16, 64)), _full((1, 64)), _full((64, 64)), _full((1, 64)),
            _full((1, 64)), _full((64, 64)), _full((1, 64)),
            _full((512, 32)), _full((32, 32)),
            _full((16, 512)), _full((32, 512)),
        ],
        out_specs=(pl.BlockSpec((BE, 128), lambda i: (i, 0)),
                   pl.BlockSpec((BE, 64), lambda i: (i, 0))),
        out_shape=(jax.ShapeDtypeStruct((E, 128), f32),
                   jax.ShapeDtypeStruct((E, 64), f32)),
    )
    rmask = jnp.kron(jnp.eye(16, dtype=f32), jnp.ones((1, 32), f32))
    tmask = jnp.tile(jnp.eye(32, dtype=f32), (1, 16))
    v0, ew1 = v0_call(g0a, g0b, edge_features, W_ep, b_ep.reshape(1, 64),
                      W_fij_0, be_0.reshape(1, 64), attn_0.reshape(1, 64),
                      W_fij_1, be_1.reshape(1, 64), wef2, bmat, rmask, tmask)

    zeros = jnp.zeros((ROWS_PER_TILE, 128), f32)
    p0 = _make_scatter_add(128)(v0, dst, zeros)

    tsrc1, tdst1, ef_nodes = pl.pallas_call(
        _tc_node1,
        out_shape=(jax.ShapeDtypeStruct((NP, 128), f32),
                   jax.ShapeDtypeStruct((NP, 128), f32),
                   jax.ShapeDtypeStruct((NP, 32), f32)),
    )(p0, b_nn.reshape(1, 32), W_ni_1, W_nj_1, W_node_1)

    g1a, g1b = _make_gather2(128, 128)(tsrc1, src, tdst1, dst)

    v1 = pl.pallas_call(
        _tc_edge1,
        grid=(GE,),
        in_specs=[
            pl.BlockSpec((BE, 128), lambda i: (i, 0)),
            pl.BlockSpec((BE, 128), lambda i: (i, 0)),
            pl.BlockSpec((BE, 64), lambda i: (i, 0)),
            _full((1, 64)),
        ],
        out_specs=pl.BlockSpec((BE, 128), lambda i: (i, 0)),
        out_shape=jax.ShapeDtypeStruct((E, 128), f32),
    )(g1a, g1b, ew1, attn_1.reshape(1, 64))

    p1 = _make_scatter_add(128)(v1, dst, zeros)

    nf, gf = pl.pallas_call(
        _tc_final,
        out_shape=(jax.ShapeDtypeStruct((N, 128), f32),
                   jax.ShapeDtypeStruct((1, 128), f32)),
    )(p1, ef_nodes, W_gate, b_gate.reshape(1, 1))

    return nf, gf
